# baseline (speedup 1.0000x reference)
import dataclasses
import functools
import jax
import jax.numpy as jnp
from jax import lax
from jax.experimental import pallas as pl
from jax.experimental.pallas import tpu as pltpu
from jax.experimental.pallas import tpu_sc as plsc

N = 10000
E = 320000
R = 4
F_IN = 128
NHID = 128
D = 256
NUM_CLASSES = 16

NC = 2
NS = 16
L = 16

CH = 128
U = 2
EP = 32 * CH * 80
NP = 10240
NBINS = 49152
HROWS = NBINS // 128

_mesh = plsc.VectorSubcoreMesh(core_axis_name="c", subcore_axis_name="s")
_HIGH = lax.Precision.HIGHEST

_SC_CP = pltpu.CompilerParams()
if "needs_layout_passes" in pltpu.CompilerParams.__dataclass_fields__:
    _SC_CP = dataclasses.replace(_SC_CP, needs_layout_passes=False)


def _prep_kernel(dst_hbm, et_hbm, src_hbm, g_hbm, dc_hbm, w_hbm,
                 dv, ev, sv, hist, iv, gv, dcv, wv, acc, sem):
    cid = lax.axis_index("c")
    sid = lax.axis_index("s")
    wid = sid * NC + cid
    iota = lax.iota(jnp.int32, L)

    @pl.loop(0, HROWS)
    def _(r):
        @pl.loop(0, 128, step=L)
        def _(j):
            hist[r, pl.ds(j, L)] = jnp.zeros((L,), jnp.float32)

    @pl.when(sid == 0)
    def _():
        pltpu.sync_copy(hist, acc)

    plsc.subcore_barrier()

    hbase = sid * (EP // NS)

    @pl.loop(0, EP // NS, step=CH)
    def _(i):
        pltpu.sync_copy(dst_hbm.at[pl.ds(hbase + i, CH)], dv)
        pltpu.sync_copy(et_hbm.at[pl.ds(hbase + i, CH)], ev)

        @pl.loop(0, CH, step=L)
        def _(j):
            key = dv[pl.ds(j, L)] * R + ev[pl.ds(j, L)]
            plsc.addupdate_scatter(hist, [key >> 7, key & 127],
                                   jnp.ones((L,), jnp.float32))

    for r0 in (0, 128, 256):
        @pl.loop(0, CH, step=L)
        def _(j, r0=r0):
            iv[pl.ds(j, L)] = iota + (r0 + j)
        pltpu.sync_copy(hist.at[pl.ds(r0, CH)], acc.at[iv], add=True)

    plsc.subcore_barrier()

    pltpu.sync_copy(acc, hist)

    base = wid * (EP // (NC * NS))

    @pl.loop(0, EP // (NC * NS), step=CH)
    def _(i):
        pltpu.sync_copy(dst_hbm.at[pl.ds(base + i, CH)], dv)
        pltpu.sync_copy(et_hbm.at[pl.ds(base + i, CH)], ev)
        pltpu.sync_copy(src_hbm.at[pl.ds(base + i, CH)], sv)

        @pl.loop(0, CH, step=L)
        def _(j):
            d = dv[pl.ds(j, L)]
            e = ev[pl.ds(j, L)]
            s = sv[pl.ds(j, L)]
            key = d * R + e
            cnt = plsc.load_gather(hist, [key >> 7, key & 127])
            valid = jnp.where(d < N, jnp.float32(1.0), jnp.float32(0.0))
            wv[pl.ds(j, L)] = valid / jnp.maximum(cnt, 1.0)
            gv[pl.ds(j, L)] = e * N + s
            dcv[pl.ds(j, L)] = jnp.minimum(d, N - 1)

        pltpu.sync_copy(gv, g_hbm.at[pl.ds(base + i, CH)])
        pltpu.sync_copy(dcv, dc_hbm.at[pl.ds(base + i, CH)])
        pltpu.sync_copy(wv, w_hbm.at[pl.ds(base + i, CH)])


def _prep(dst_p, et_p, src_p):
    f = pl.kernel(
        _prep_kernel,
        out_type=[jax.ShapeDtypeStruct((EP,), jnp.int32),
                  jax.ShapeDtypeStruct((EP,), jnp.int32),
                  jax.ShapeDtypeStruct((EP,), jnp.float32)],
        mesh=_mesh,
        compiler_params=_SC_CP,
        scratch_types=[pltpu.VMEM((CH,), jnp.int32),
                       pltpu.VMEM((CH,), jnp.int32),
                       pltpu.VMEM((CH,), jnp.int32),
                       pltpu.VMEM((HROWS, 128), jnp.float32),
                       pltpu.VMEM((CH,), jnp.int32),
                       pltpu.VMEM((CH,), jnp.int32),
                       pltpu.VMEM((CH,), jnp.int32),
                       pltpu.VMEM((CH,), jnp.float32),
                       pltpu.VMEM_SHARED((HROWS, 128), jnp.float32),
                       pltpu.SemaphoreType.DMA],
    )
    return f(dst_p, et_p, src_p)


def _agg_kernel(col_split, y_hbm, g_hbm, dc_hbm, w_hbm, part_hbm,
                gv, gv2, dv, ws, rows, acc, gsem, ssem):
    cid = lax.axis_index("c")
    sid = lax.axis_index("s")

    @pl.loop(0, CH)
    def _(r):
        @pl.loop(0, 128, step=L)
        def _(j):
            rows[r, pl.ds(j, L)] = jnp.zeros((L,), jnp.float32)

    rpt = NP // NS
    @pl.loop(0, rpt, step=CH)
    def _(r0):
        pltpu.sync_copy(rows, acc.at[pl.ds(sid * rpt + r0, CH)])

    plsc.subcore_barrier()

    if col_split:
        nloc = EP // NS
        base = sid * nloc
        goff = cid * (R * N)
    else:
        nloc = EP // (NC * NS)
        base = (sid * NC + cid) * nloc
        goff = None

    @pl.loop(0, nloc, step=CH)
    def _(i):
        pltpu.sync_copy(g_hbm.at[pl.ds(base + i, CH)], gv)
        pltpu.sync_copy(dc_hbm.at[pl.ds(base + i, CH)], dv)
        pltpu.sync_copy(w_hbm.at[pl.ds(base + i, CH)], ws)
        if col_split:
            @pl.loop(0, CH, step=L)
            def _(j):
                gv2[pl.ds(j, L)] = gv[pl.ds(j, L)] + goff
            idx = gv2
        else:
            idx = gv
        pltpu.async_copy(y_hbm.at[idx], rows, gsem).wait()

        @pl.loop(0, CH)
        def _(j):
            jv = jnp.zeros((L,), jnp.int32) + j
            wj = plsc.load_gather(ws, [jv])

            @pl.loop(0, 128, step=L)
            def _(k):
                rows[j, pl.ds(k, L)] = rows[j, pl.ds(k, L)] * wj

        pltpu.sync_copy(rows, acc.at[dv], add=True)

    plsc.subcore_barrier()

    @pl.loop(0, rpt, step=CH)
    def _(r0):
        r = sid * rpt + r0
        pltpu.sync_copy(acc.at[pl.ds(r, CH)],
                        part_hbm.at[pl.ds(cid * NP + r, CH)])


def _agg(y_flat, g2d, d2d, w2d, col_split):
    f = pl.kernel(
        functools.partial(_agg_kernel, col_split),
        out_type=jax.ShapeDtypeStruct((2 * NP, 128), jnp.float32),
        mesh=_mesh,
        compiler_params=_SC_CP,
        scratch_types=[pltpu.VMEM((CH,), jnp.int32),
                       pltpu.VMEM((CH,), jnp.int32),
                       pltpu.VMEM((CH,), jnp.int32),
                       pltpu.VMEM((CH,), jnp.float32),
                       pltpu.VMEM((CH, 128), jnp.float32),
                       pltpu.VMEM_SHARED((NP, 128), jnp.float32),
                       pltpu.SemaphoreType.DMA,
                       pltpu.SemaphoreType.DMA],
    )
    return f(y_flat, g2d, d2d, w2d)


BN = 400


def _table1_kernel(x_ref, w_ref, root_ref, y_ref, root_out_ref):
    x = x_ref[...]
    for r in range(R):
        y_ref[r] = jnp.dot(x, w_ref[r], precision=_HIGH)
    root_out_ref[...] = jnp.dot(x, root_ref[...], precision=_HIGH)


def _table1(x, w, root):
    fdim, odim = root.shape
    grid = (N // BN,)
    return pl.pallas_call(
        _table1_kernel,
        grid=grid,
        in_specs=[pl.BlockSpec((BN, fdim), lambda i: (i, 0)),
                  pl.BlockSpec((R, fdim, odim), lambda i: (0, 0, 0)),
                  pl.BlockSpec((fdim, odim), lambda i: (0, 0))],
        out_specs=[pl.BlockSpec((R, BN, odim), lambda i: (0, i, 0)),
                   pl.BlockSpec((BN, odim), lambda i: (i, 0))],
        out_shape=[jax.ShapeDtypeStruct((R, N, odim), jnp.float32),
                   jax.ShapeDtypeStruct((N, odim), jnp.float32)],
    )(x, w, root)


def _table2_kernel(xm_ref, h_ref, mu_ref, w_ref, root_ref, y_ref, root_out_ref):
    xm = xm_ref[...]
    se = h_ref[...] + mu_ref[...]
    f = xm.shape[1]
    for r in range(R):
        y_ref[r] = (jnp.dot(xm, w_ref[r, :f], precision=_HIGH)
                    + jnp.dot(se, w_ref[r, f:], precision=_HIGH))
    root_out_ref[...] = (jnp.dot(xm, root_ref[:f], precision=_HIGH)
                         + jnp.dot(se, root_ref[f:], precision=_HIGH))


def _table2(xm, h, mu, w, root):
    f2, odim = root.shape
    f = xm.shape[1]
    grid = (N // BN,)
    return pl.pallas_call(
        _table2_kernel,
        grid=grid,
        in_specs=[pl.BlockSpec((BN, f), lambda i: (i, 0)),
                  pl.BlockSpec((BN, f2 - f), lambda i: (i, 0)),
                  pl.BlockSpec((1, f2 - f), lambda i: (0, 0)),
                  pl.BlockSpec((R, f2, odim), lambda i: (0, 0, 0)),
                  pl.BlockSpec((f2, odim), lambda i: (0, 0))],
        out_specs=[pl.BlockSpec((R, BN, odim), lambda i: (0, i, 0)),
                   pl.BlockSpec((BN, odim), lambda i: (i, 0))],
        out_shape=[jax.ShapeDtypeStruct((R, N, odim), jnp.float32),
                   jax.ShapeDtypeStruct((N, odim), jnp.float32)],
    )(xm, h, mu, w, root)


def _post_kernel(col_split, p0_ref, p1_ref, root_ref, b_ref, out_ref, mu_ref,
                 accum):
    i = pl.program_id(0)
    if col_split:
        v = jnp.concatenate([p0_ref[...], p1_ref[...]], axis=1)
    else:
        v = p0_ref[...] + p1_ref[...]
    v = v + root_ref[...] + b_ref[...]
    out = jnp.where(v >= 0, v, 0.01 * v)
    out_ref[...] = out

    @pl.when(i == 0)
    def _():
        accum[...] = jnp.zeros_like(accum)

    accum[...] += jnp.sum(out, axis=0, keepdims=True)

    @pl.when(i == pl.num_programs(0) - 1)
    def _():
        mu_ref[...] = accum[...] * (1.0 / N)


def _post_kernel3(col_split, p_ref0, p_ref1, root_ref, b_ref, out_ref,
                  mu_ref, accum):
    _post_kernel(col_split, p_ref0.at[0], p_ref1.at[0], root_ref, b_ref,
                 out_ref, mu_ref, accum)


def _post(part, root_out, bias, col_split):
    odim = root_out.shape[1]
    grid = (N // BN,)
    part3 = part.reshape(2, NP, 128)
    return pl.pallas_call(
        functools.partial(_post_kernel3, col_split),
        grid=grid,
        in_specs=[pl.BlockSpec((1, BN, 128), lambda i: (0, i, 0)),
                  pl.BlockSpec((1, BN, 128), lambda i: (1, i, 0)),
                  pl.BlockSpec((BN, odim), lambda i: (i, 0)),
                  pl.BlockSpec((1, odim), lambda i: (0, 0))],
        out_specs=[pl.BlockSpec((BN, odim), lambda i: (i, 0)),
                   pl.BlockSpec((1, odim), lambda i: (0, 0))],
        out_shape=[jax.ShapeDtypeStruct((N, odim), jnp.float32),
                   jax.ShapeDtypeStruct((1, odim), jnp.float32)],
        scratch_shapes=[pltpu.VMEM((1, odim), jnp.float32)],
    )(part3, part3, root_out, bias.reshape(1, odim))


def _ln_rows(v, g, b):
    mu = jnp.mean(v, axis=-1, keepdims=True)
    var = jnp.mean((v - mu) * (v - mu), axis=-1, keepdims=True)
    return (v - mu) * jax.lax.rsqrt(var + 1e-5) * g + b


def _final_kernel(qd_ref, wv_ref, bv_ref, wo_ref, bo_ref, fc_ref, fb_ref,
                  lng_ref, lnb_ref, ow_ref, ob_ref, out_ref, accum):
    i = pl.program_id(0)
    q = qd_ref[...]
    v = jnp.dot(q, wv_ref[...].T, precision=_HIGH) + bv_ref[...]
    a = jnp.dot(v, wo_ref[...].T, precision=_HIGH) + bo_ref[...]
    t = _ln_rows(a + q, lng_ref[...], lnb_ref[...])
    ff = jnp.dot(t, fc_ref[...].T, precision=_HIGH) + fb_ref[...]
    o = _ln_rows(ff + t, lng_ref[...], lnb_ref[...])

    @pl.when(i == 0)
    def _():
        accum[...] = jnp.zeros_like(accum)

    accum[...] += jnp.sum(o, axis=0, keepdims=True)

    @pl.when(i == pl.num_programs(0) - 1)
    def _():
        pooled = accum[...] * (1.0 / N)
        out_ref[...] = jnp.dot(pooled, ow_ref[...].T,
                               precision=_HIGH) + ob_ref[...]


def _final(qd, Wv, bv, Wo, bo, fc_w, fc_b, ln_g, ln_b, out_w, out_b):
    grid = (N // BN,)
    return pl.pallas_call(
        _final_kernel,
        grid=grid,
        in_specs=[pl.BlockSpec((BN, D), lambda i: (i, 0)),
                  pl.BlockSpec((D, D), lambda i: (0, 0)),
                  pl.BlockSpec((1, D), lambda i: (0, 0)),
                  pl.BlockSpec((D, D), lambda i: (0, 0)),
                  pl.BlockSpec((1, D), lambda i: (0, 0)),
                  pl.BlockSpec((D, D), lambda i: (0, 0)),
                  pl.BlockSpec((1, D), lambda i: (0, 0)),
                  pl.BlockSpec((1, D), lambda i: (0, 0)),
                  pl.BlockSpec((1, D), lambda i: (0, 0)),
                  pl.BlockSpec((NUM_CLASSES, D), lambda i: (0, 0)),
                  pl.BlockSpec((1, NUM_CLASSES), lambda i: (0, 0))],
        out_specs=pl.BlockSpec((1, NUM_CLASSES), lambda i: (0, 0)),
        out_shape=jax.ShapeDtypeStruct((1, NUM_CLASSES), jnp.float32),
        scratch_shapes=[pltpu.VMEM((1, D), jnp.float32)],
    )(qd, Wv, bv.reshape(1, D), Wo, bo.reshape(1, D), fc_w,
      fc_b.reshape(1, D), ln_g.reshape(1, D), ln_b.reshape(1, D),
      out_w, out_b.reshape(1, NUM_CLASSES))


def _rgcn_layer(xin, parts, w, root, bias, g, dc, wgt, concat_parts=None):
    odim = root.shape[1]
    col_split = odim == 256
    if concat_parts is None:
        y, root_out = _table1(xin, w, root)
    else:
        h, mu = concat_parts
        y, root_out = _table2(xin, h, mu, w, root)
    if col_split:
        y_flat = jnp.concatenate(
            [y[:, :, :128].reshape(R * N, 128),
             y[:, :, 128:].reshape(R * N, 128)], axis=0)
    else:
        y_flat = y.reshape(R * N, 128)
    part = _agg(y_flat, g, dc, wgt, col_split)
    out, mu = _post(part, root_out, bias, col_split)
    return out, mu


def kernel(x, edge_index, edge_attr, sW1, sroot1, sb1, sW2, sroot2, sb2,
           sW3, sroot3, sb3, fW1, froot1, fb1, fW2, froot2, fb2, fW3,
           froot3, fb3, fW4, froot4, fb4, Wq, bq, Wk, bk, Wv, bv, Wo, bo,
           enc_fc_w, enc_fc_b, ln_g, ln_b, out_w, out_b):
    src = edge_index[0]
    dst = edge_index[1]
    et = edge_attr
    pad = EP - E
    src_p = jnp.pad(src, (0, pad))
    dst_p = jnp.pad(dst, (0, pad), constant_values=N)
    et_p = jnp.pad(et, (0, pad))

    g, dc, wgt = _prep(dst_p, et_p, src_p)

    h1, mu1 = _rgcn_layer(x, None, sW1, sroot1, sb1, g, dc, wgt)
    h2, mu2 = _rgcn_layer(h1, None, sW2, sroot2, sb2, g, dc, wgt)
    h3, mu3 = _rgcn_layer(h2, None, sW3, sroot3, sb3, g, dc, wgt)

    xm1, _ = _rgcn_layer(x, None, fW1, froot1, fb1, g, dc, wgt)
    xm2, _ = _rgcn_layer(xm1, None, fW2, froot2, fb2, g, dc, wgt,
                         concat_parts=(h1, mu1))
    xm3, _ = _rgcn_layer(xm2, None, fW3, froot3, fb3, g, dc, wgt,
                         concat_parts=(h2, mu2))
    qd, _ = _rgcn_layer(xm3, None, fW4, froot4, fb4, g, dc, wgt,
                        concat_parts=(h3, mu3))

    return _final(qd, Wv, bv, Wo, bo, enc_fc_w, enc_fc_b, ln_g, ln_b,
                  out_w, out_b)


# speedup vs baseline: 1.3205x; 1.3205x over previous
import dataclasses
import functools
import jax
import jax.numpy as jnp
from jax import lax
from jax.experimental import pallas as pl
from jax.experimental.pallas import tpu as pltpu
from jax.experimental.pallas import tpu_sc as plsc

N = 10000
E = 320000
R = 4
F_IN = 128
NHID = 128
D = 256
NUM_CLASSES = 16

NC = 2
NS = 16
L = 16

CH = 128
EP = 32 * CH * 79
NP = 10240
NBINS = 49152
HROWS = NBINS // 128

_mesh = plsc.VectorSubcoreMesh(core_axis_name="c", subcore_axis_name="s")
_HIGH = lax.Precision.HIGHEST

_SC_CP = pltpu.CompilerParams()
if "needs_layout_passes" in pltpu.CompilerParams.__dataclass_fields__:
    _SC_CP = dataclasses.replace(_SC_CP, needs_layout_passes=False)


def _prep_kernel(dst_hbm, et_hbm, src_hbm, g_hbm, dc_hbm, w_hbm,
                 dv, ev, sv, hist, iv, gv, dcv, wv, acc, sem):
    cid = lax.axis_index("c")
    sid = lax.axis_index("s")
    wid = sid * NC + cid
    iota = lax.iota(jnp.int32, L)

    @pl.loop(0, HROWS)
    def _(r):
        @pl.loop(0, 128, step=L)
        def _(j):
            hist[r, pl.ds(j, L)] = jnp.zeros((L,), jnp.float32)

    @pl.when(sid == 0)
    def _():
        pltpu.sync_copy(hist, acc)

    plsc.subcore_barrier()

    hbase = sid * (EP // NS)

    @pl.loop(0, EP // NS, step=CH)
    def _(i):
        pltpu.sync_copy(dst_hbm.at[pl.ds(hbase + i, CH)], dv)
        pltpu.sync_copy(et_hbm.at[pl.ds(hbase + i, CH)], ev)

        @pl.loop(0, CH, step=L)
        def _(j):
            key = dv[pl.ds(j, L)] * R + ev[pl.ds(j, L)]
            plsc.addupdate_scatter(hist, [key >> 7, key & 127],
                                   jnp.ones((L,), jnp.float32))

    for r0 in (0, 128, 256):
        @pl.loop(0, CH, step=L)
        def _(j, r0=r0):
            iv[pl.ds(j, L)] = iota + (r0 + j)
        pltpu.sync_copy(hist.at[pl.ds(r0, CH)], acc.at[iv], add=True)

    plsc.subcore_barrier()

    pltpu.sync_copy(acc, hist)

    base = wid * (EP // (NC * NS))

    @pl.loop(0, EP // (NC * NS), step=CH)
    def _(i):
        pltpu.sync_copy(dst_hbm.at[pl.ds(base + i, CH)], dv)
        pltpu.sync_copy(et_hbm.at[pl.ds(base + i, CH)], ev)
        pltpu.sync_copy(src_hbm.at[pl.ds(base + i, CH)], sv)

        @pl.loop(0, CH, step=L)
        def _(j):
            d = dv[pl.ds(j, L)]
            e = ev[pl.ds(j, L)]
            s = sv[pl.ds(j, L)]
            key = d * R + e
            cnt = plsc.load_gather(hist, [key >> 7, key & 127])
            valid = jnp.where(d < N, jnp.float32(1.0), jnp.float32(0.0))
            wv[pl.ds(j, L)] = valid / jnp.maximum(cnt, 1.0)
            gv[pl.ds(j, L)] = e * N + s
            dcv[pl.ds(j, L)] = jnp.minimum(d, N - 1)

        pltpu.sync_copy(gv, g_hbm.at[pl.ds(base + i, CH)])
        pltpu.sync_copy(dcv, dc_hbm.at[pl.ds(base + i, CH)])
        pltpu.sync_copy(wv, w_hbm.at[pl.ds(base + i, CH)])


def _prep(dst_p, et_p, src_p):
    f = pl.kernel(
        _prep_kernel,
        out_type=[jax.ShapeDtypeStruct((EP,), jnp.int32),
                  jax.ShapeDtypeStruct((EP,), jnp.int32),
                  jax.ShapeDtypeStruct((EP,), jnp.float32)],
        mesh=_mesh,
        compiler_params=_SC_CP,
        scratch_types=[pltpu.VMEM((CH,), jnp.int32),
                       pltpu.VMEM((CH,), jnp.int32),
                       pltpu.VMEM((CH,), jnp.int32),
                       pltpu.VMEM((HROWS, 128), jnp.float32),
                       pltpu.VMEM((CH,), jnp.int32),
                       pltpu.VMEM((CH,), jnp.int32),
                       pltpu.VMEM((CH,), jnp.int32),
                       pltpu.VMEM((CH,), jnp.float32),
                       pltpu.VMEM_SHARED((HROWS, 128), jnp.float32),
                       pltpu.SemaphoreType.DMA],
    )
    return f(dst_p, et_p, src_p)


def _agg_kernel(col_split, y_hbm, g_hbm, dc_hbm, w_hbm, part_hbm,
                gv, gv2, dv, rows, zbuf, ws, acc, gsem):
    cid = lax.axis_index("c")
    sid = lax.axis_index("s")

    @pl.loop(0, CH)
    def _(r):
        @pl.loop(0, 128, step=L)
        def _(j):
            zbuf[r, pl.ds(j, L)] = jnp.zeros((L,), jnp.float32)

    rpt = NP // NS
    @pl.loop(0, rpt, step=CH)
    def _(r0):
        pltpu.sync_copy(zbuf, acc.at[pl.ds(sid * rpt + r0, CH)])

    plsc.subcore_barrier()

    if col_split:
        nloc = EP // NS
        base = sid * nloc
        goff = cid * (R * N)
    else:
        nloc = EP // (NC * NS)
        base = (sid * NC + cid) * nloc
        goff = None

    @pl.loop(0, nloc, step=CH)
    def _(i):
        pltpu.sync_copy(g_hbm.at[pl.ds(base + i, CH)], gv)
        pltpu.sync_copy(dc_hbm.at[pl.ds(base + i, CH)], dv)
        pltpu.sync_copy(w_hbm.at[pl.ds(base + i, CH)], ws)
        if col_split:
            @pl.loop(0, CH, step=L)
            def _(j):
                gv2[pl.ds(j, L)] = gv[pl.ds(j, L)] + goff
            idx = gv2
        else:
            idx = gv
        pltpu.async_copy(y_hbm.at[idx], rows, gsem).wait()

        @pl.loop(0, CH)
        def _(j):
            jv = jnp.zeros((L,), jnp.int32) + j
            wj = plsc.load_gather(ws, [jv])

            @pl.loop(0, 128, step=L)
            def _(k):
                rows[j, pl.ds(k, L)] = rows[j, pl.ds(k, L)] * wj

        pltpu.sync_copy(rows, acc.at[dv], add=True)

    plsc.subcore_barrier()

    @pl.loop(0, rpt, step=CH)
    def _(r0):
        r = sid * rpt + r0
        pltpu.sync_copy(acc.at[pl.ds(r, CH)],
                        part_hbm.at[pl.ds(cid * NP + r, CH)])


def _agg(y_flat, g2d, d2d, w2d, col_split):
    f = pl.kernel(
        functools.partial(_agg_kernel, col_split),
        out_type=jax.ShapeDtypeStruct((2 * NP, 128), jnp.float32),
        mesh=_mesh,
        compiler_params=_SC_CP,
        scratch_types=[pltpu.VMEM((CH,), jnp.int32),
                       pltpu.VMEM((CH,), jnp.int32),
                       pltpu.VMEM((CH,), jnp.int32),
                       pltpu.VMEM((CH, 128), jnp.float32),
                       pltpu.VMEM((128, 128), jnp.float32),
                       pltpu.VMEM((CH,), jnp.float32),
                       pltpu.VMEM_SHARED((NP, 128), jnp.float32),
                       pltpu.SemaphoreType.DMA],
    )
    return f(y_flat, g2d, d2d, w2d)


BN = 400


def _table1_kernel(x_ref, w_ref, root_ref, y_ref, root_out_ref):
    x = x_ref[...]
    for r in range(R):
        y_ref[r] = jnp.dot(x, w_ref[r], precision=_HIGH)
    root_out_ref[...] = jnp.dot(x, root_ref[...], precision=_HIGH)


def _table1(x, w, root):
    fdim, odim = root.shape
    grid = (N // BN,)
    return pl.pallas_call(
        _table1_kernel,
        grid=grid,
        in_specs=[pl.BlockSpec((BN, fdim), lambda i: (i, 0)),
                  pl.BlockSpec((R, fdim, odim), lambda i: (0, 0, 0)),
                  pl.BlockSpec((fdim, odim), lambda i: (0, 0))],
        out_specs=[pl.BlockSpec((R, BN, odim), lambda i: (0, i, 0)),
                   pl.BlockSpec((BN, odim), lambda i: (i, 0))],
        out_shape=[jax.ShapeDtypeStruct((R, N, odim), jnp.float32),
                   jax.ShapeDtypeStruct((N, odim), jnp.float32)],
    )(x, w, root)


def _table2_kernel(xm_ref, h_ref, mu_ref, w_ref, root_ref, y_ref, root_out_ref):
    xm = xm_ref[...]
    se = h_ref[...] + mu_ref[...]
    f = xm.shape[1]
    for r in range(R):
        y_ref[r] = (jnp.dot(xm, w_ref[r, :f], precision=_HIGH)
                    + jnp.dot(se, w_ref[r, f:], precision=_HIGH))
    root_out_ref[...] = (jnp.dot(xm, root_ref[:f], precision=_HIGH)
                         + jnp.dot(se, root_ref[f:], precision=_HIGH))


def _table2(xm, h, mu, w, root):
    f2, odim = root.shape
    f = xm.shape[1]
    grid = (N // BN,)
    return pl.pallas_call(
        _table2_kernel,
        grid=grid,
        in_specs=[pl.BlockSpec((BN, f), lambda i: (i, 0)),
                  pl.BlockSpec((BN, f2 - f), lambda i: (i, 0)),
                  pl.BlockSpec((1, f2 - f), lambda i: (0, 0)),
                  pl.BlockSpec((R, f2, odim), lambda i: (0, 0, 0)),
                  pl.BlockSpec((f2, odim), lambda i: (0, 0))],
        out_specs=[pl.BlockSpec((R, BN, odim), lambda i: (0, i, 0)),
                   pl.BlockSpec((BN, odim), lambda i: (i, 0))],
        out_shape=[jax.ShapeDtypeStruct((R, N, odim), jnp.float32),
                   jax.ShapeDtypeStruct((N, odim), jnp.float32)],
    )(xm, h, mu, w, root)


def _post_kernel(col_split, p0_ref, p1_ref, root_ref, b_ref, out_ref, mu_ref,
                 accum):
    i = pl.program_id(0)
    if col_split:
        v = jnp.concatenate([p0_ref[...], p1_ref[...]], axis=1)
    else:
        v = p0_ref[...] + p1_ref[...]
    v = v + root_ref[...] + b_ref[...]
    out = jnp.where(v >= 0, v, 0.01 * v)
    out_ref[...] = out

    @pl.when(i == 0)
    def _():
        accum[...] = jnp.zeros_like(accum)

    accum[...] += jnp.sum(out, axis=0, keepdims=True)

    @pl.when(i == pl.num_programs(0) - 1)
    def _():
        mu_ref[...] = accum[...] * (1.0 / N)


def _post_kernel3(col_split, p_ref0, p_ref1, root_ref, b_ref, out_ref,
                  mu_ref, accum):
    _post_kernel(col_split, p_ref0.at[0], p_ref1.at[0], root_ref, b_ref,
                 out_ref, mu_ref, accum)


def _post(part, root_out, bias, col_split):
    odim = root_out.shape[1]
    grid = (N // BN,)
    part3 = part.reshape(2, NP, 128)
    return pl.pallas_call(
        functools.partial(_post_kernel3, col_split),
        grid=grid,
        in_specs=[pl.BlockSpec((1, BN, 128), lambda i: (0, i, 0)),
                  pl.BlockSpec((1, BN, 128), lambda i: (1, i, 0)),
                  pl.BlockSpec((BN, odim), lambda i: (i, 0)),
                  pl.BlockSpec((1, odim), lambda i: (0, 0))],
        out_specs=[pl.BlockSpec((BN, odim), lambda i: (i, 0)),
                   pl.BlockSpec((1, odim), lambda i: (0, 0))],
        out_shape=[jax.ShapeDtypeStruct((N, odim), jnp.float32),
                   jax.ShapeDtypeStruct((1, odim), jnp.float32)],
        scratch_shapes=[pltpu.VMEM((1, odim), jnp.float32)],
    )(part3, part3, root_out, bias.reshape(1, odim))


def _ln_rows(v, g, b):
    mu = jnp.mean(v, axis=-1, keepdims=True)
    var = jnp.mean((v - mu) * (v - mu), axis=-1, keepdims=True)
    return (v - mu) * jax.lax.rsqrt(var + 1e-5) * g + b


def _final_kernel(qd_ref, wv_ref, bv_ref, wo_ref, bo_ref, fc_ref, fb_ref,
                  lng_ref, lnb_ref, ow_ref, ob_ref, out_ref, accum):
    i = pl.program_id(0)
    q = qd_ref[...]
    v = jnp.dot(q, wv_ref[...].T, precision=_HIGH) + bv_ref[...]
    a = jnp.dot(v, wo_ref[...].T, precision=_HIGH) + bo_ref[...]
    t = _ln_rows(a + q, lng_ref[...], lnb_ref[...])
    ff = jnp.dot(t, fc_ref[...].T, precision=_HIGH) + fb_ref[...]
    o = _ln_rows(ff + t, lng_ref[...], lnb_ref[...])

    @pl.when(i == 0)
    def _():
        accum[...] = jnp.zeros_like(accum)

    accum[...] += jnp.sum(o, axis=0, keepdims=True)

    @pl.when(i == pl.num_programs(0) - 1)
    def _():
        pooled = accum[...] * (1.0 / N)
        out_ref[...] = jnp.dot(pooled, ow_ref[...].T,
                               precision=_HIGH) + ob_ref[...]


def _final(qd, Wv, bv, Wo, bo, fc_w, fc_b, ln_g, ln_b, out_w, out_b):
    grid = (N // BN,)
    return pl.pallas_call(
        _final_kernel,
        grid=grid,
        in_specs=[pl.BlockSpec((BN, D), lambda i: (i, 0)),
                  pl.BlockSpec((D, D), lambda i: (0, 0)),
                  pl.BlockSpec((1, D), lambda i: (0, 0)),
                  pl.BlockSpec((D, D), lambda i: (0, 0)),
                  pl.BlockSpec((1, D), lambda i: (0, 0)),
                  pl.BlockSpec((D, D), lambda i: (0, 0)),
                  pl.BlockSpec((1, D), lambda i: (0, 0)),
                  pl.BlockSpec((1, D), lambda i: (0, 0)),
                  pl.BlockSpec((1, D), lambda i: (0, 0)),
                  pl.BlockSpec((NUM_CLASSES, D), lambda i: (0, 0)),
                  pl.BlockSpec((1, NUM_CLASSES), lambda i: (0, 0))],
        out_specs=pl.BlockSpec((1, NUM_CLASSES), lambda i: (0, 0)),
        out_shape=jax.ShapeDtypeStruct((1, NUM_CLASSES), jnp.float32),
        scratch_shapes=[pltpu.VMEM((1, D), jnp.float32)],
    )(qd, Wv, bv.reshape(1, D), Wo, bo.reshape(1, D), fc_w,
      fc_b.reshape(1, D), ln_g.reshape(1, D), ln_b.reshape(1, D),
      out_w, out_b.reshape(1, NUM_CLASSES))


def _rgcn_layer(xin, parts, w, root, bias, g, dc, wgt, concat_parts=None):
    odim = root.shape[1]
    col_split = odim == 256
    if concat_parts is None:
        y, root_out = _table1(xin, w, root)
    else:
        h, mu = concat_parts
        y, root_out = _table2(xin, h, mu, w, root)
    if col_split:
        y_flat = jnp.concatenate(
            [y[:, :, :128].reshape(R * N, 128),
             y[:, :, 128:].reshape(R * N, 128)], axis=0)
    else:
        y_flat = y.reshape(R * N, 128)
    part = _agg(y_flat, g, dc, wgt, col_split)
    out, mu = _post(part, root_out, bias, col_split)
    return out, mu


def kernel(x, edge_index, edge_attr, sW1, sroot1, sb1, sW2, sroot2, sb2,
           sW3, sroot3, sb3, fW1, froot1, fb1, fW2, froot2, fb2, fW3,
           froot3, fb3, fW4, froot4, fb4, Wq, bq, Wk, bk, Wv, bv, Wo, bo,
           enc_fc_w, enc_fc_b, ln_g, ln_b, out_w, out_b):
    src = edge_index[0]
    dst = edge_index[1]
    et = edge_attr
    pad = EP - E
    src_p = jnp.pad(src, (0, pad))
    dst_p = jnp.pad(dst, (0, pad), constant_values=N)
    et_p = jnp.pad(et, (0, pad))

    g, dc, wgt = _prep(dst_p, et_p, src_p)

    h1, mu1 = _rgcn_layer(x, None, sW1, sroot1, sb1, g, dc, wgt)
    h2, mu2 = _rgcn_layer(h1, None, sW2, sroot2, sb2, g, dc, wgt)
    h3, mu3 = _rgcn_layer(h2, None, sW3, sroot3, sb3, g, dc, wgt)

    xm1, _ = _rgcn_layer(x, None, fW1, froot1, fb1, g, dc, wgt)
    xm2, _ = _rgcn_layer(xm1, None, fW2, froot2, fb2, g, dc, wgt,
                         concat_parts=(h1, mu1))
    xm3, _ = _rgcn_layer(xm2, None, fW3, froot3, fb3, g, dc, wgt,
                         concat_parts=(h2, mu2))
    qd, _ = _rgcn_layer(xm3, None, fW4, froot4, fb4, g, dc, wgt,
                        concat_parts=(h3, mu3))

    return _final(qd, Wv, bv, Wo, bo, enc_fc_w, enc_fc_b, ln_g, ln_b,
                  out_w, out_b)


# speedup vs baseline: 1.6177x; 1.2251x over previous
import dataclasses
import functools
import jax
import jax.numpy as jnp
from jax import lax
from jax.experimental import pallas as pl
from jax.experimental.pallas import tpu as pltpu
from jax.experimental.pallas import tpu_sc as plsc

N = 10000
E = 320000
R = 4
F_IN = 128
NHID = 128
D = 256
NUM_CLASSES = 16

NC = 2
NS = 16
L = 16

CH = 128
EP = 32 * CH * 79
NP = 10240
NBINS = 49152
HROWS = NBINS // 128

_mesh = plsc.VectorSubcoreMesh(core_axis_name="c", subcore_axis_name="s")
_HIGH = lax.Precision.HIGHEST

_SC_CP = pltpu.CompilerParams()
if "needs_layout_passes" in pltpu.CompilerParams.__dataclass_fields__:
    _SC_CP = dataclasses.replace(_SC_CP, needs_layout_passes=False)


def _prep_kernel(dst_hbm, et_hbm, src_hbm, g_hbm, dc_hbm, w_hbm,
                 dv, ev, sv, hist, iv, gv, dcv, wv, acc, sem):
    cid = lax.axis_index("c")
    sid = lax.axis_index("s")
    wid = sid * NC + cid
    iota = lax.iota(jnp.int32, L)

    @pl.loop(0, HROWS)
    def _(r):
        @pl.loop(0, 128, step=L)
        def _(j):
            hist[r, pl.ds(j, L)] = jnp.zeros((L,), jnp.float32)

    @pl.when(sid == 0)
    def _():
        pltpu.sync_copy(hist, acc)

    plsc.subcore_barrier()

    hbase = sid * (EP // NS)

    @pl.loop(0, EP // NS, step=CH)
    def _(i):
        pltpu.sync_copy(dst_hbm.at[pl.ds(hbase + i, CH)], dv)
        pltpu.sync_copy(et_hbm.at[pl.ds(hbase + i, CH)], ev)

        @pl.loop(0, CH, step=L)
        def _(j):
            key = dv[pl.ds(j, L)] * R + ev[pl.ds(j, L)]
            plsc.addupdate_scatter(hist, [key >> 7, key & 127],
                                   jnp.ones((L,), jnp.float32))

    for r0 in (0, 128, 256):
        @pl.loop(0, CH, step=L)
        def _(j, r0=r0):
            iv[pl.ds(j, L)] = iota + (r0 + j)
        pltpu.sync_copy(hist.at[pl.ds(r0, CH)], acc.at[iv], add=True)

    plsc.subcore_barrier()

    pltpu.sync_copy(acc, hist)

    base = wid * (EP // (NC * NS))

    @pl.loop(0, EP // (NC * NS), step=CH)
    def _(i):
        pltpu.sync_copy(dst_hbm.at[pl.ds(base + i, CH)], dv)
        pltpu.sync_copy(et_hbm.at[pl.ds(base + i, CH)], ev)
        pltpu.sync_copy(src_hbm.at[pl.ds(base + i, CH)], sv)

        @pl.loop(0, CH, step=L)
        def _(j):
            d = dv[pl.ds(j, L)]
            e = ev[pl.ds(j, L)]
            s = sv[pl.ds(j, L)]
            key = d * R + e
            cnt = plsc.load_gather(hist, [key >> 7, key & 127])
            valid = jnp.where(d < N, jnp.float32(1.0), jnp.float32(0.0))
            wv[pl.ds(j, L)] = valid / jnp.maximum(cnt, 1.0)
            gv[pl.ds(j, L)] = e * N + s
            dcv[pl.ds(j, L)] = jnp.minimum(d, N - 1)

        pltpu.sync_copy(gv, g_hbm.at[pl.ds(base + i, CH)])
        pltpu.sync_copy(dcv, dc_hbm.at[pl.ds(base + i, CH)])
        pltpu.sync_copy(wv, w_hbm.at[pl.ds(base + i, CH)])


def _prep(dst_p, et_p, src_p):
    f = pl.kernel(
        _prep_kernel,
        out_type=[jax.ShapeDtypeStruct((EP,), jnp.int32),
                  jax.ShapeDtypeStruct((EP,), jnp.int32),
                  jax.ShapeDtypeStruct((EP,), jnp.float32)],
        mesh=_mesh,
        compiler_params=_SC_CP,
        scratch_types=[pltpu.VMEM((CH,), jnp.int32),
                       pltpu.VMEM((CH,), jnp.int32),
                       pltpu.VMEM((CH,), jnp.int32),
                       pltpu.VMEM((HROWS, 128), jnp.float32),
                       pltpu.VMEM((CH,), jnp.int32),
                       pltpu.VMEM((CH,), jnp.int32),
                       pltpu.VMEM((CH,), jnp.int32),
                       pltpu.VMEM((CH,), jnp.float32),
                       pltpu.VMEM_SHARED((HROWS, 128), jnp.float32),
                       pltpu.SemaphoreType.DMA],
    )
    return f(dst_p, et_p, src_p)


def _agg_kernel(col_split, y_hbm, g_hbm, dc_hbm, w_hbm, part_hbm,
                gvA, gvB, dvA, dvB, wsA, wsB, rowsA, rowsB, acc,
                semA, semB):
    cid = lax.axis_index("c")
    sid = lax.axis_index("s")
    bankA = (gvA, dvA, wsA, rowsA, semA)
    bankB = (gvB, dvB, wsB, rowsB, semB)

    @pl.loop(0, CH)
    def _(r):
        @pl.loop(0, 128, step=L)
        def _(j):
            rowsA[r, pl.ds(j, L)] = jnp.zeros((L,), jnp.float32)

    rpt = NP // NS
    @pl.loop(0, rpt, step=CH)
    def _(r0):
        pltpu.sync_copy(rowsA, acc.at[pl.ds(sid * rpt + r0, CH)])

    plsc.subcore_barrier()

    if col_split:
        nchunk = EP // CH // NS
        base = sid * nchunk * CH
        goff = cid * (R * N)
    else:
        nchunk = EP // CH // (NC * NS)
        base = (sid * NC + cid) * nchunk * CH
        goff = None

    def load_and_fire(bank, off):
        gv, dv, ws, rows, sem = bank
        pltpu.sync_copy(g_hbm.at[pl.ds(off, CH)], gv)
        pltpu.sync_copy(dc_hbm.at[pl.ds(off, CH)], dv)
        pltpu.sync_copy(w_hbm.at[pl.ds(off, CH)], ws)
        if col_split:
            @pl.loop(0, CH, step=L)
            def _(j):
                gv[pl.ds(j, L)] = gv[pl.ds(j, L)] + goff
        return pltpu.async_copy(y_hbm.at[gv], rows, sem)

    def drain(bank, copy):
        _, dv, ws, rows, _ = bank
        copy.wait()

        @pl.loop(0, CH)
        def _(j):
            jv = jnp.zeros((L,), jnp.int32) + j
            wj = plsc.load_gather(ws, [jv])
            for k in range(0, 128, L):
                rows[j, pl.ds(k, L)] = rows[j, pl.ds(k, L)] * wj

        pltpu.sync_copy(rows, acc.at[dv], add=True)

    npairs = nchunk // 2

    @pl.loop(0, npairs)
    def _(s):
        off = base + (2 * s) * CH
        cA = load_and_fire(bankA, off)
        cB = load_and_fire(bankB, off + CH)
        drain(bankA, cA)
        drain(bankB, cB)

    if nchunk % 2:
        cA = load_and_fire(bankA, base + (nchunk - 1) * CH)
        drain(bankA, cA)

    plsc.subcore_barrier()

    @pl.loop(0, rpt, step=CH)
    def _(r0):
        r = sid * rpt + r0
        pltpu.sync_copy(acc.at[pl.ds(r, CH)],
                        part_hbm.at[pl.ds(cid * NP + r, CH)])


def _agg(y_flat, g2d, d2d, w2d, col_split):
    f = pl.kernel(
        functools.partial(_agg_kernel, col_split),
        out_type=jax.ShapeDtypeStruct((2 * NP, 128), jnp.float32),
        mesh=_mesh,
        compiler_params=_SC_CP,
        scratch_types=[pltpu.VMEM((CH,), jnp.int32),
                       pltpu.VMEM((CH,), jnp.int32),
                       pltpu.VMEM((CH,), jnp.int32),
                       pltpu.VMEM((CH,), jnp.int32),
                       pltpu.VMEM((CH,), jnp.float32),
                       pltpu.VMEM((CH,), jnp.float32),
                       pltpu.VMEM((CH, 128), jnp.float32),
                       pltpu.VMEM((CH, 128), jnp.float32),
                       pltpu.VMEM_SHARED((NP, 128), jnp.float32),
                       pltpu.SemaphoreType.DMA,
                       pltpu.SemaphoreType.DMA],
    )
    return f(y_flat, g2d, d2d, w2d)


BN = 400


def _table1_kernel(x_ref, w_ref, root_ref, y_ref, root_out_ref):
    x = x_ref[...]
    for r in range(R):
        y_ref[r] = jnp.dot(x, w_ref[r], precision=_HIGH)
    root_out_ref[...] = jnp.dot(x, root_ref[...], precision=_HIGH)


def _table1(x, w, root):
    fdim, odim = root.shape
    grid = (N // BN,)
    return pl.pallas_call(
        _table1_kernel,
        grid=grid,
        in_specs=[pl.BlockSpec((BN, fdim), lambda i: (i, 0)),
                  pl.BlockSpec((R, fdim, odim), lambda i: (0, 0, 0)),
                  pl.BlockSpec((fdim, odim), lambda i: (0, 0))],
        out_specs=[pl.BlockSpec((R, BN, odim), lambda i: (0, i, 0)),
                   pl.BlockSpec((BN, odim), lambda i: (i, 0))],
        out_shape=[jax.ShapeDtypeStruct((R, N, odim), jnp.float32),
                   jax.ShapeDtypeStruct((N, odim), jnp.float32)],
    )(x, w, root)


def _table2_kernel(xm_ref, h_ref, mu_ref, w_ref, root_ref, y_ref, root_out_ref):
    xm = xm_ref[...]
    se = h_ref[...] + mu_ref[...]
    f = xm.shape[1]
    for r in range(R):
        y_ref[r] = (jnp.dot(xm, w_ref[r, :f], precision=_HIGH)
                    + jnp.dot(se, w_ref[r, f:], precision=_HIGH))
    root_out_ref[...] = (jnp.dot(xm, root_ref[:f], precision=_HIGH)
                         + jnp.dot(se, root_ref[f:], precision=_HIGH))


def _table2(xm, h, mu, w, root):
    f2, odim = root.shape
    f = xm.shape[1]
    grid = (N // BN,)
    return pl.pallas_call(
        _table2_kernel,
        grid=grid,
        in_specs=[pl.BlockSpec((BN, f), lambda i: (i, 0)),
                  pl.BlockSpec((BN, f2 - f), lambda i: (i, 0)),
                  pl.BlockSpec((1, f2 - f), lambda i: (0, 0)),
                  pl.BlockSpec((R, f2, odim), lambda i: (0, 0, 0)),
                  pl.BlockSpec((f2, odim), lambda i: (0, 0))],
        out_specs=[pl.BlockSpec((R, BN, odim), lambda i: (0, i, 0)),
                   pl.BlockSpec((BN, odim), lambda i: (i, 0))],
        out_shape=[jax.ShapeDtypeStruct((R, N, odim), jnp.float32),
                   jax.ShapeDtypeStruct((N, odim), jnp.float32)],
    )(xm, h, mu, w, root)


def _post_kernel(col_split, p0_ref, p1_ref, root_ref, b_ref, out_ref, mu_ref,
                 accum):
    i = pl.program_id(0)
    if col_split:
        v = jnp.concatenate([p0_ref[...], p1_ref[...]], axis=1)
    else:
        v = p0_ref[...] + p1_ref[...]
    v = v + root_ref[...] + b_ref[...]
    out = jnp.where(v >= 0, v, 0.01 * v)
    out_ref[...] = out

    @pl.when(i == 0)
    def _():
        accum[...] = jnp.zeros_like(accum)

    accum[...] += jnp.sum(out, axis=0, keepdims=True)

    @pl.when(i == pl.num_programs(0) - 1)
    def _():
        mu_ref[...] = accum[...] * (1.0 / N)


def _post_kernel3(col_split, p_ref0, p_ref1, root_ref, b_ref, out_ref,
                  mu_ref, accum):
    _post_kernel(col_split, p_ref0.at[0], p_ref1.at[0], root_ref, b_ref,
                 out_ref, mu_ref, accum)


def _post(part, root_out, bias, col_split):
    odim = root_out.shape[1]
    grid = (N // BN,)
    part3 = part.reshape(2, NP, 128)
    return pl.pallas_call(
        functools.partial(_post_kernel3, col_split),
        grid=grid,
        in_specs=[pl.BlockSpec((1, BN, 128), lambda i: (0, i, 0)),
                  pl.BlockSpec((1, BN, 128), lambda i: (1, i, 0)),
                  pl.BlockSpec((BN, odim), lambda i: (i, 0)),
                  pl.BlockSpec((1, odim), lambda i: (0, 0))],
        out_specs=[pl.BlockSpec((BN, odim), lambda i: (i, 0)),
                   pl.BlockSpec((1, odim), lambda i: (0, 0))],
        out_shape=[jax.ShapeDtypeStruct((N, odim), jnp.float32),
                   jax.ShapeDtypeStruct((1, odim), jnp.float32)],
        scratch_shapes=[pltpu.VMEM((1, odim), jnp.float32)],
    )(part3, part3, root_out, bias.reshape(1, odim))


def _ln_rows(v, g, b):
    mu = jnp.mean(v, axis=-1, keepdims=True)
    var = jnp.mean((v - mu) * (v - mu), axis=-1, keepdims=True)
    return (v - mu) * jax.lax.rsqrt(var + 1e-5) * g + b


def _final_kernel(qd_ref, wv_ref, bv_ref, wo_ref, bo_ref, fc_ref, fb_ref,
                  lng_ref, lnb_ref, ow_ref, ob_ref, out_ref, accum):
    i = pl.program_id(0)
    q = qd_ref[...]
    v = jnp.dot(q, wv_ref[...].T, precision=_HIGH) + bv_ref[...]
    a = jnp.dot(v, wo_ref[...].T, precision=_HIGH) + bo_ref[...]
    t = _ln_rows(a + q, lng_ref[...], lnb_ref[...])
    ff = jnp.dot(t, fc_ref[...].T, precision=_HIGH) + fb_ref[...]
    o = _ln_rows(ff + t, lng_ref[...], lnb_ref[...])

    @pl.when(i == 0)
    def _():
        accum[...] = jnp.zeros_like(accum)

    accum[...] += jnp.sum(o, axis=0, keepdims=True)

    @pl.when(i == pl.num_programs(0) - 1)
    def _():
        pooled = accum[...] * (1.0 / N)
        out_ref[...] = jnp.dot(pooled, ow_ref[...].T,
                               precision=_HIGH) + ob_ref[...]


def _final(qd, Wv, bv, Wo, bo, fc_w, fc_b, ln_g, ln_b, out_w, out_b):
    grid = (N // BN,)
    return pl.pallas_call(
        _final_kernel,
        grid=grid,
        in_specs=[pl.BlockSpec((BN, D), lambda i: (i, 0)),
                  pl.BlockSpec((D, D), lambda i: (0, 0)),
                  pl.BlockSpec((1, D), lambda i: (0, 0)),
                  pl.BlockSpec((D, D), lambda i: (0, 0)),
                  pl.BlockSpec((1, D), lambda i: (0, 0)),
                  pl.BlockSpec((D, D), lambda i: (0, 0)),
                  pl.BlockSpec((1, D), lambda i: (0, 0)),
                  pl.BlockSpec((1, D), lambda i: (0, 0)),
                  pl.BlockSpec((1, D), lambda i: (0, 0)),
                  pl.BlockSpec((NUM_CLASSES, D), lambda i: (0, 0)),
                  pl.BlockSpec((1, NUM_CLASSES), lambda i: (0, 0))],
        out_specs=pl.BlockSpec((1, NUM_CLASSES), lambda i: (0, 0)),
        out_shape=jax.ShapeDtypeStruct((1, NUM_CLASSES), jnp.float32),
        scratch_shapes=[pltpu.VMEM((1, D), jnp.float32)],
    )(qd, Wv, bv.reshape(1, D), Wo, bo.reshape(1, D), fc_w,
      fc_b.reshape(1, D), ln_g.reshape(1, D), ln_b.reshape(1, D),
      out_w, out_b.reshape(1, NUM_CLASSES))


def _rgcn_layer(xin, parts, w, root, bias, g, dc, wgt, concat_parts=None):
    odim = root.shape[1]
    col_split = odim == 256
    if concat_parts is None:
        y, root_out = _table1(xin, w, root)
    else:
        h, mu = concat_parts
        y, root_out = _table2(xin, h, mu, w, root)
    if col_split:
        y_flat = jnp.concatenate(
            [y[:, :, :128].reshape(R * N, 128),
             y[:, :, 128:].reshape(R * N, 128)], axis=0)
    else:
        y_flat = y.reshape(R * N, 128)
    part = _agg(y_flat, g, dc, wgt, col_split)
    out, mu = _post(part, root_out, bias, col_split)
    return out, mu


def kernel(x, edge_index, edge_attr, sW1, sroot1, sb1, sW2, sroot2, sb2,
           sW3, sroot3, sb3, fW1, froot1, fb1, fW2, froot2, fb2, fW3,
           froot3, fb3, fW4, froot4, fb4, Wq, bq, Wk, bk, Wv, bv, Wo, bo,
           enc_fc_w, enc_fc_b, ln_g, ln_b, out_w, out_b):
    src = edge_index[0]
    dst = edge_index[1]
    et = edge_attr
    pad = EP - E
    src_p = jnp.pad(src, (0, pad))
    dst_p = jnp.pad(dst, (0, pad), constant_values=N)
    et_p = jnp.pad(et, (0, pad))

    g, dc, wgt = _prep(dst_p, et_p, src_p)

    h1, mu1 = _rgcn_layer(x, None, sW1, sroot1, sb1, g, dc, wgt)
    h2, mu2 = _rgcn_layer(h1, None, sW2, sroot2, sb2, g, dc, wgt)
    h3, mu3 = _rgcn_layer(h2, None, sW3, sroot3, sb3, g, dc, wgt)

    xm1, _ = _rgcn_layer(x, None, fW1, froot1, fb1, g, dc, wgt)
    xm2, _ = _rgcn_layer(xm1, None, fW2, froot2, fb2, g, dc, wgt,
                         concat_parts=(h1, mu1))
    xm3, _ = _rgcn_layer(xm2, None, fW3, froot3, fb3, g, dc, wgt,
                         concat_parts=(h2, mu2))
    qd, _ = _rgcn_layer(xm3, None, fW4, froot4, fb4, g, dc, wgt,
                        concat_parts=(h3, mu3))

    return _final(qd, Wv, bv, Wo, bo, enc_fc_w, enc_fc_b, ln_g, ln_b,
                  out_w, out_b)


# speedup vs baseline: 1.7822x; 1.1017x over previous
import dataclasses
import functools
import jax
import jax.numpy as jnp
from jax import lax
from jax.experimental import pallas as pl
from jax.experimental.pallas import tpu as pltpu
from jax.experimental.pallas import tpu_sc as plsc

N = 10000
E = 320000
R = 4
F_IN = 128
NHID = 128
D = 256
NUM_CLASSES = 16

NC = 2
NS = 16
L = 16

CH = 128
EP = 32 * CH * 79
NP = 10240
NBINS = 49152
HROWS = NBINS // 128

_mesh = plsc.VectorSubcoreMesh(core_axis_name="c", subcore_axis_name="s")
_HIGH = lax.Precision.HIGHEST

_SC_CP = pltpu.CompilerParams()
if "needs_layout_passes" in pltpu.CompilerParams.__dataclass_fields__:
    _SC_CP = dataclasses.replace(_SC_CP, needs_layout_passes=False)


def _prep_kernel(dst_hbm, et_hbm, src_hbm, g_hbm, dc_hbm, w_hbm,
                 dv, ev, sv, hist, iv, gv, dcv, wv, acc, sem):
    cid = lax.axis_index("c")
    sid = lax.axis_index("s")
    wid = sid * NC + cid
    iota = lax.iota(jnp.int32, L)

    @pl.loop(0, HROWS)
    def _(r):
        @pl.loop(0, 128, step=L)
        def _(j):
            hist[r, pl.ds(j, L)] = jnp.zeros((L,), jnp.float32)

    @pl.when(sid == 0)
    def _():
        pltpu.sync_copy(hist, acc)

    plsc.subcore_barrier()

    hbase = sid * (EP // NS)

    @pl.loop(0, EP // NS, step=CH)
    def _(i):
        pltpu.sync_copy(dst_hbm.at[pl.ds(hbase + i, CH)], dv)
        pltpu.sync_copy(et_hbm.at[pl.ds(hbase + i, CH)], ev)

        @pl.loop(0, CH, step=L)
        def _(j):
            key = dv[pl.ds(j, L)] * R + ev[pl.ds(j, L)]
            plsc.addupdate_scatter(hist, [key >> 7, key & 127],
                                   jnp.ones((L,), jnp.float32))

    for r0 in (0, 128, 256):
        @pl.loop(0, CH, step=L)
        def _(j, r0=r0):
            iv[pl.ds(j, L)] = iota + (r0 + j)
        pltpu.sync_copy(hist.at[pl.ds(r0, CH)], acc.at[iv], add=True)

    plsc.subcore_barrier()

    pltpu.sync_copy(acc, hist)

    base = wid * (EP // (NC * NS))

    @pl.loop(0, EP // (NC * NS), step=CH)
    def _(i):
        pltpu.sync_copy(dst_hbm.at[pl.ds(base + i, CH)], dv)
        pltpu.sync_copy(et_hbm.at[pl.ds(base + i, CH)], ev)
        pltpu.sync_copy(src_hbm.at[pl.ds(base + i, CH)], sv)

        @pl.loop(0, CH, step=L)
        def _(j):
            d = dv[pl.ds(j, L)]
            e = ev[pl.ds(j, L)]
            s = sv[pl.ds(j, L)]
            key = d * R + e
            cnt = plsc.load_gather(hist, [key >> 7, key & 127])
            valid = jnp.where(d < N, jnp.float32(1.0), jnp.float32(0.0))
            wv[pl.ds(j, L)] = valid / jnp.maximum(cnt, 1.0)
            gv[pl.ds(j, L)] = e * N + s
            dcv[pl.ds(j, L)] = jnp.minimum(d, N - 1)

        pltpu.sync_copy(gv, g_hbm.at[pl.ds(base + i, CH)])
        pltpu.sync_copy(dcv, dc_hbm.at[pl.ds(base + i, CH)])
        pltpu.sync_copy(wv, w_hbm.at[pl.ds(base + i, CH)])


def _prep(dst_p, et_p, src_p):
    f = pl.kernel(
        _prep_kernel,
        out_type=[jax.ShapeDtypeStruct((EP,), jnp.int32),
                  jax.ShapeDtypeStruct((EP,), jnp.int32),
                  jax.ShapeDtypeStruct((EP,), jnp.float32)],
        mesh=_mesh,
        compiler_params=_SC_CP,
        scratch_types=[pltpu.VMEM((CH,), jnp.int32),
                       pltpu.VMEM((CH,), jnp.int32),
                       pltpu.VMEM((CH,), jnp.int32),
                       pltpu.VMEM((HROWS, 128), jnp.float32),
                       pltpu.VMEM((CH,), jnp.int32),
                       pltpu.VMEM((CH,), jnp.int32),
                       pltpu.VMEM((CH,), jnp.int32),
                       pltpu.VMEM((CH,), jnp.float32),
                       pltpu.VMEM_SHARED((HROWS, 128), jnp.float32),
                       pltpu.SemaphoreType.DMA],
    )
    return f(dst_p, et_p, src_p)


def _agg_kernel(col_split, y_hbm, g_hbm, dc_hbm, w_hbm, part_hbm,
                gvA, gvB, dvA, dvB, wsA, wsB, rowsA, rowsB, acc,
                semA, semB):
    cid = lax.axis_index("c")
    sid = lax.axis_index("s")
    bankA = (gvA, dvA, wsA, rowsA, semA)
    bankB = (gvB, dvB, wsB, rowsB, semB)

    @pl.loop(0, CH)
    def _(r):
        @pl.loop(0, 128, step=L)
        def _(j):
            rowsA[r, pl.ds(j, L)] = jnp.zeros((L,), jnp.float32)

    rpt = NP // NS
    @pl.loop(0, rpt, step=CH)
    def _(r0):
        pltpu.sync_copy(rowsA, acc.at[pl.ds(sid * rpt + r0, CH)])

    plsc.subcore_barrier()

    if col_split:
        nchunk = EP // CH // NS
        base = sid * nchunk * CH
        goff = cid * (R * N)
    else:
        nchunk = EP // CH // (NC * NS)
        base = (sid * NC + cid) * nchunk * CH
        goff = None

    def load_and_fire(bank, off):
        gv, dv, ws, rows, sem = bank
        pltpu.sync_copy(g_hbm.at[pl.ds(off, CH)], gv)
        pltpu.sync_copy(dc_hbm.at[pl.ds(off, CH)], dv)
        pltpu.sync_copy(w_hbm.at[pl.ds(off, CH)], ws)
        if col_split:
            @pl.loop(0, CH, step=L)
            def _(j):
                gv[pl.ds(j, L)] = gv[pl.ds(j, L)] + goff
        return pltpu.async_copy(y_hbm.at[gv], rows, sem)

    def drain_ready(bank):
        _, dv, ws, rows, _ = bank

        @pl.loop(0, CH)
        def _(j):
            jv = jnp.zeros((L,), jnp.int32) + j
            wj = plsc.load_gather(ws, [jv])
            for k in range(0, 128, L):
                rows[j, pl.ds(k, L)] = rows[j, pl.ds(k, L)] * wj

        pltpu.sync_copy(rows, acc.at[dv], add=True)

    def wait_bytes(bank):
        _, _, _, rows, sem = bank
        pltpu.make_async_copy(y_hbm.at[pl.ds(0, CH)], rows, sem).wait()

    T = (nchunk - 1) // 2
    load_and_fire(bankA, base)

    @pl.loop(0, T)
    def _(s):
        off = base + (2 * s + 1) * CH
        load_and_fire(bankB, off)
        wait_bytes(bankA)
        drain_ready(bankA)
        load_and_fire(bankA, off + CH)
        wait_bytes(bankB)
        drain_ready(bankB)

    wait_bytes(bankA)
    drain_ready(bankA)
    if nchunk % 2 == 0:
        load_and_fire(bankB, base + (nchunk - 1) * CH)
        wait_bytes(bankB)
        drain_ready(bankB)

    plsc.subcore_barrier()

    @pl.loop(0, rpt, step=CH)
    def _(r0):
        r = sid * rpt + r0
        pltpu.sync_copy(acc.at[pl.ds(r, CH)],
                        part_hbm.at[pl.ds(cid * NP + r, CH)])


def _agg(y_flat, g2d, d2d, w2d, col_split):
    f = pl.kernel(
        functools.partial(_agg_kernel, col_split),
        out_type=jax.ShapeDtypeStruct((2 * NP, 128), jnp.float32),
        mesh=_mesh,
        compiler_params=_SC_CP,
        scratch_types=[pltpu.VMEM((CH,), jnp.int32),
                       pltpu.VMEM((CH,), jnp.int32),
                       pltpu.VMEM((CH,), jnp.int32),
                       pltpu.VMEM((CH,), jnp.int32),
                       pltpu.VMEM((CH,), jnp.float32),
                       pltpu.VMEM((CH,), jnp.float32),
                       pltpu.VMEM((CH, 128), jnp.float32),
                       pltpu.VMEM((CH, 128), jnp.float32),
                       pltpu.VMEM_SHARED((NP, 128), jnp.float32),
                       pltpu.SemaphoreType.DMA,
                       pltpu.SemaphoreType.DMA],
    )
    return f(y_flat, g2d, d2d, w2d)


BN = 400


def _table1_kernel(x_ref, w_ref, root_ref, y_ref, root_out_ref):
    x = x_ref[...]
    for r in range(R):
        y_ref[r] = jnp.dot(x, w_ref[r], precision=_HIGH)
    root_out_ref[...] = jnp.dot(x, root_ref[...], precision=_HIGH)


def _table1(x, w, root):
    fdim, odim = root.shape
    grid = (N // BN,)
    return pl.pallas_call(
        _table1_kernel,
        grid=grid,
        in_specs=[pl.BlockSpec((BN, fdim), lambda i: (i, 0)),
                  pl.BlockSpec((R, fdim, odim), lambda i: (0, 0, 0)),
                  pl.BlockSpec((fdim, odim), lambda i: (0, 0))],
        out_specs=[pl.BlockSpec((R, BN, odim), lambda i: (0, i, 0)),
                   pl.BlockSpec((BN, odim), lambda i: (i, 0))],
        out_shape=[jax.ShapeDtypeStruct((R, N, odim), jnp.float32),
                   jax.ShapeDtypeStruct((N, odim), jnp.float32)],
    )(x, w, root)


def _table2_kernel(xm_ref, h_ref, mu_ref, w_ref, root_ref, y_ref, root_out_ref):
    xm = xm_ref[...]
    se = h_ref[...] + mu_ref[...]
    f = xm.shape[1]
    for r in range(R):
        y_ref[r] = (jnp.dot(xm, w_ref[r, :f], precision=_HIGH)
                    + jnp.dot(se, w_ref[r, f:], precision=_HIGH))
    root_out_ref[...] = (jnp.dot(xm, root_ref[:f], precision=_HIGH)
                         + jnp.dot(se, root_ref[f:], precision=_HIGH))


def _table2(xm, h, mu, w, root):
    f2, odim = root.shape
    f = xm.shape[1]
    grid = (N // BN,)
    return pl.pallas_call(
        _table2_kernel,
        grid=grid,
        in_specs=[pl.BlockSpec((BN, f), lambda i: (i, 0)),
                  pl.BlockSpec((BN, f2 - f), lambda i: (i, 0)),
                  pl.BlockSpec((1, f2 - f), lambda i: (0, 0)),
                  pl.BlockSpec((R, f2, odim), lambda i: (0, 0, 0)),
                  pl.BlockSpec((f2, odim), lambda i: (0, 0))],
        out_specs=[pl.BlockSpec((R, BN, odim), lambda i: (0, i, 0)),
                   pl.BlockSpec((BN, odim), lambda i: (i, 0))],
        out_shape=[jax.ShapeDtypeStruct((R, N, odim), jnp.float32),
                   jax.ShapeDtypeStruct((N, odim), jnp.float32)],
    )(xm, h, mu, w, root)


def _post_kernel(col_split, p0_ref, p1_ref, root_ref, b_ref, out_ref, mu_ref,
                 accum):
    i = pl.program_id(0)
    if col_split:
        v = jnp.concatenate([p0_ref[...], p1_ref[...]], axis=1)
    else:
        v = p0_ref[...] + p1_ref[...]
    v = v + root_ref[...] + b_ref[...]
    out = jnp.where(v >= 0, v, 0.01 * v)
    out_ref[...] = out

    @pl.when(i == 0)
    def _():
        accum[...] = jnp.zeros_like(accum)

    accum[...] += jnp.sum(out, axis=0, keepdims=True)

    @pl.when(i == pl.num_programs(0) - 1)
    def _():
        mu_ref[...] = accum[...] * (1.0 / N)


def _post_kernel3(col_split, p_ref0, p_ref1, root_ref, b_ref, out_ref,
                  mu_ref, accum):
    _post_kernel(col_split, p_ref0.at[0], p_ref1.at[0], root_ref, b_ref,
                 out_ref, mu_ref, accum)


def _post(part, root_out, bias, col_split):
    odim = root_out.shape[1]
    grid = (N // BN,)
    part3 = part.reshape(2, NP, 128)
    return pl.pallas_call(
        functools.partial(_post_kernel3, col_split),
        grid=grid,
        in_specs=[pl.BlockSpec((1, BN, 128), lambda i: (0, i, 0)),
                  pl.BlockSpec((1, BN, 128), lambda i: (1, i, 0)),
                  pl.BlockSpec((BN, odim), lambda i: (i, 0)),
                  pl.BlockSpec((1, odim), lambda i: (0, 0))],
        out_specs=[pl.BlockSpec((BN, odim), lambda i: (i, 0)),
                   pl.BlockSpec((1, odim), lambda i: (0, 0))],
        out_shape=[jax.ShapeDtypeStruct((N, odim), jnp.float32),
                   jax.ShapeDtypeStruct((1, odim), jnp.float32)],
        scratch_shapes=[pltpu.VMEM((1, odim), jnp.float32)],
    )(part3, part3, root_out, bias.reshape(1, odim))


def _ln_rows(v, g, b):
    mu = jnp.mean(v, axis=-1, keepdims=True)
    var = jnp.mean((v - mu) * (v - mu), axis=-1, keepdims=True)
    return (v - mu) * jax.lax.rsqrt(var + 1e-5) * g + b


def _final_kernel(qd_ref, wv_ref, bv_ref, wo_ref, bo_ref, fc_ref, fb_ref,
                  lng_ref, lnb_ref, ow_ref, ob_ref, out_ref, accum):
    i = pl.program_id(0)
    q = qd_ref[...]
    v = jnp.dot(q, wv_ref[...].T, precision=_HIGH) + bv_ref[...]
    a = jnp.dot(v, wo_ref[...].T, precision=_HIGH) + bo_ref[...]
    t = _ln_rows(a + q, lng_ref[...], lnb_ref[...])
    ff = jnp.dot(t, fc_ref[...].T, precision=_HIGH) + fb_ref[...]
    o = _ln_rows(ff + t, lng_ref[...], lnb_ref[...])

    @pl.when(i == 0)
    def _():
        accum[...] = jnp.zeros_like(accum)

    accum[...] += jnp.sum(o, axis=0, keepdims=True)

    @pl.when(i == pl.num_programs(0) - 1)
    def _():
        pooled = accum[...] * (1.0 / N)
        out_ref[...] = jnp.dot(pooled, ow_ref[...].T,
                               precision=_HIGH) + ob_ref[...]


def _final(qd, Wv, bv, Wo, bo, fc_w, fc_b, ln_g, ln_b, out_w, out_b):
    grid = (N // BN,)
    return pl.pallas_call(
        _final_kernel,
        grid=grid,
        in_specs=[pl.BlockSpec((BN, D), lambda i: (i, 0)),
                  pl.BlockSpec((D, D), lambda i: (0, 0)),
                  pl.BlockSpec((1, D), lambda i: (0, 0)),
                  pl.BlockSpec((D, D), lambda i: (0, 0)),
                  pl.BlockSpec((1, D), lambda i: (0, 0)),
                  pl.BlockSpec((D, D), lambda i: (0, 0)),
                  pl.BlockSpec((1, D), lambda i: (0, 0)),
                  pl.BlockSpec((1, D), lambda i: (0, 0)),
                  pl.BlockSpec((1, D), lambda i: (0, 0)),
                  pl.BlockSpec((NUM_CLASSES, D), lambda i: (0, 0)),
                  pl.BlockSpec((1, NUM_CLASSES), lambda i: (0, 0))],
        out_specs=pl.BlockSpec((1, NUM_CLASSES), lambda i: (0, 0)),
        out_shape=jax.ShapeDtypeStruct((1, NUM_CLASSES), jnp.float32),
        scratch_shapes=[pltpu.VMEM((1, D), jnp.float32)],
    )(qd, Wv, bv.reshape(1, D), Wo, bo.reshape(1, D), fc_w,
      fc_b.reshape(1, D), ln_g.reshape(1, D), ln_b.reshape(1, D),
      out_w, out_b.reshape(1, NUM_CLASSES))


def _rgcn_layer(xin, parts, w, root, bias, g, dc, wgt, concat_parts=None):
    odim = root.shape[1]
    col_split = odim == 256
    if concat_parts is None:
        y, root_out = _table1(xin, w, root)
    else:
        h, mu = concat_parts
        y, root_out = _table2(xin, h, mu, w, root)
    if col_split:
        y_flat = jnp.concatenate(
            [y[:, :, :128].reshape(R * N, 128),
             y[:, :, 128:].reshape(R * N, 128)], axis=0)
    else:
        y_flat = y.reshape(R * N, 128)
    part = _agg(y_flat, g, dc, wgt, col_split)
    out, mu = _post(part, root_out, bias, col_split)
    return out, mu


def kernel(x, edge_index, edge_attr, sW1, sroot1, sb1, sW2, sroot2, sb2,
           sW3, sroot3, sb3, fW1, froot1, fb1, fW2, froot2, fb2, fW3,
           froot3, fb3, fW4, froot4, fb4, Wq, bq, Wk, bk, Wv, bv, Wo, bo,
           enc_fc_w, enc_fc_b, ln_g, ln_b, out_w, out_b):
    src = edge_index[0]
    dst = edge_index[1]
    et = edge_attr
    pad = EP - E
    src_p = jnp.pad(src, (0, pad))
    dst_p = jnp.pad(dst, (0, pad), constant_values=N)
    et_p = jnp.pad(et, (0, pad))

    g, dc, wgt = _prep(dst_p, et_p, src_p)

    h1, mu1 = _rgcn_layer(x, None, sW1, sroot1, sb1, g, dc, wgt)
    h2, mu2 = _rgcn_layer(h1, None, sW2, sroot2, sb2, g, dc, wgt)
    h3, mu3 = _rgcn_layer(h2, None, sW3, sroot3, sb3, g, dc, wgt)

    xm1, _ = _rgcn_layer(x, None, fW1, froot1, fb1, g, dc, wgt)
    xm2, _ = _rgcn_layer(xm1, None, fW2, froot2, fb2, g, dc, wgt,
                         concat_parts=(h1, mu1))
    xm3, _ = _rgcn_layer(xm2, None, fW3, froot3, fb3, g, dc, wgt,
                         concat_parts=(h2, mu2))
    qd, _ = _rgcn_layer(xm3, None, fW4, froot4, fb4, g, dc, wgt,
                        concat_parts=(h3, mu3))

    return _final(qd, Wv, bv, Wo, bo, enc_fc_w, enc_fc_b, ln_g, ln_b,
                  out_w, out_b)
